# Initial kernel scaffold; baseline (speedup 1.0000x reference)
import jax
import jax.numpy as jnp
from jax.experimental import pallas as pl


def kernel(position, edge_src, edge_dst, fc1_w1, fc1_w2, W1a, W1b, fc2_w1, fc2_w2, W2a, W2b):
    raise NotImplementedError("write your pallas kernel here")


# speedup vs baseline: 1.7996x; 1.7996x over previous
import functools
import math

import jax
import jax.numpy as jnp
from jax import lax
from jax.experimental import pallas as pl
from jax.experimental.pallas import tpu as pltpu
from jax.experimental.pallas import tpu_sc as plsc

N = 10000
E = 160000
NC = 2
NS = 16
NW = NC * NS
EPW = E // NW
CH = 128
NFULL = EPW // CH
TAIL = EPW - NFULL * CH
RPT = N // NS
INV = 1.0 / math.sqrt(3.8)
KSOH = 1.14136 * math.exp(2.0) * math.sqrt(3.0)


def _mesh():
    return plsc.VectorSubcoreMesh(core_axis_name="c", subcore_axis_name="s",
                                  num_cores=NC, num_subcores=NS)


def _f32(shape):
    return jax.ShapeDtypeStruct(shape, jnp.float32)


def _zero_rows(rows_ref, nrows, width):
    def body(i, _):
        for j in range(width // 16):
            rows_ref[i, pl.ds(16 * j, 16)] = jnp.zeros((16,), jnp.float32)
        return 0
    lax.fori_loop(0, nrows, body, 0)


def _init_acc(acc, rows_ref, sid, width):
    _zero_rows(rows_ref, CH, width)
    base = sid * RPT
    off = 0
    while off < RPT:
        n = min(CH, RPT - off)
        pltpu.sync_copy(rows_ref.at[pl.ds(0, n)], acc.at[pl.ds(base + off, n)])
        off += n


def _build_p1():
    def body(pos_hbm, src_hbm, dst_hbm, ev_hbm,
             idx_s, idx_d, rows_s, rows_d, idx_st, idx_dt, rows_st, rows_dt, sem):
        c = lax.axis_index("c")
        s = lax.axis_index("s")
        base = (c * NS + s) * EPW

        def chunk(off, n, i_s, i_d, r_s, r_d):
            pltpu.sync_copy(src_hbm.at[pl.ds(off, n)], i_s)
            pltpu.sync_copy(dst_hbm.at[pl.ds(off, n)], i_d)
            pltpu.async_copy(pos_hbm.at[i_s], r_s, sem).wait()
            pltpu.async_copy(pos_hbm.at[i_d], r_d, sem).wait()

            def sub(i, _):
                r_s[i, :] = r_s[i, :] - r_d[i, :]
                return 0
            lax.fori_loop(0, n, sub, 0)
            pltpu.sync_copy(r_s, ev_hbm.at[pl.ds(off, n)])

        def loop(k, _):
            chunk(base + k * CH, CH, idx_s, idx_d, rows_s, rows_d)
            return 0
        lax.fori_loop(0, NFULL, loop, 0)
        chunk(base + NFULL * CH, TAIL, idx_st, idx_dt, rows_st, rows_dt)

    return pl.kernel(
        body,
        compiler_params=pltpu.CompilerParams(use_tc_tiling_on_sc=False),
        out_type=_f32((E, 16)),
        mesh=_mesh(),
        scratch_types=[
            pltpu.VMEM((CH,), jnp.int32), pltpu.VMEM((CH,), jnp.int32),
            pltpu.VMEM((CH, 16), jnp.float32), pltpu.VMEM((CH, 16), jnp.float32),
            pltpu.VMEM((TAIL,), jnp.int32), pltpu.VMEM((TAIL,), jnp.int32),
            pltpu.VMEM((TAIL, 16), jnp.float32), pltpu.VMEM((TAIL, 16), jnp.float32),
            pltpu.SemaphoreType.DMA,
        ],
    )


def _build_scatter(D):
    def body(val_hbm, dst_hbm, out_hbm, idx_v, rows_v, idx_t, rows_t, acc):
        c = lax.axis_index("c")
        s = lax.axis_index("s")
        _init_acc(acc, rows_v, s, D)
        plsc.subcore_barrier()
        base = (c * NS + s) * EPW

        def chunk(off, n, i_v, r_v):
            pltpu.sync_copy(dst_hbm.at[pl.ds(off, n)], i_v)
            pltpu.sync_copy(val_hbm.at[pl.ds(off, n)], r_v)
            pltpu.sync_copy(r_v, acc.at[i_v], add=True)

        def loop(k, _):
            chunk(base + k * CH, CH, idx_v, rows_v)
            return 0
        lax.fori_loop(0, NFULL, loop, 0)
        chunk(base + NFULL * CH, TAIL, idx_t, rows_t)
        plsc.subcore_barrier()
        pltpu.sync_copy(acc.at[pl.ds(s * RPT, RPT)],
                        out_hbm.at[pl.ds(c * N + s * RPT, RPT)])

    return pl.kernel(
        body,
        compiler_params=pltpu.CompilerParams(use_tc_tiling_on_sc=False),
        out_type=_f32((2 * N, D)),
        mesh=_mesh(),
        scratch_types=[
            pltpu.VMEM((CH,), jnp.int32),
            pltpu.VMEM((CH, D), jnp.float32),
            pltpu.VMEM((TAIL,), jnp.int32),
            pltpu.VMEM((TAIL, D), jnp.float32),
            pltpu.VMEM_SHARED((N, D), jnp.float32),
        ],
    )


def _build_p4():
    def body(p0_hbm, p1_hbm, src_hbm, g0_hbm, g1_hbm,
             idx_v, rows_v, idx_t, rows_t, sem):
        c = lax.axis_index("c")
        s = lax.axis_index("s")
        base = (c * NS + s) * EPW

        def chunk(off, n, i_v, r_v):
            pltpu.sync_copy(src_hbm.at[pl.ds(off, n)], i_v)
            pltpu.async_copy(p0_hbm.at[i_v], r_v, sem).wait()
            pltpu.sync_copy(r_v, g0_hbm.at[pl.ds(off, n)])
            pltpu.async_copy(p1_hbm.at[i_v], r_v, sem).wait()
            pltpu.sync_copy(r_v, g1_hbm.at[pl.ds(off, n)])

        def loop(k, _):
            chunk(base + k * CH, CH, idx_v, rows_v)
            return 0
        lax.fori_loop(0, NFULL, loop, 0)
        chunk(base + NFULL * CH, TAIL, idx_t, rows_t)

    return pl.kernel(
        body,
        compiler_params=pltpu.CompilerParams(use_tc_tiling_on_sc=False),
        out_type=(_f32((E, 16)), _f32((E, 16))),
        mesh=_mesh(),
        scratch_types=[
            pltpu.VMEM((CH,), jnp.int32), pltpu.VMEM((CH, 16), jnp.float32),
            pltpu.VMEM((TAIL,), jnp.int32), pltpu.VMEM((TAIL, 16), jnp.float32),
            pltpu.SemaphoreType.DMA,
        ],
    )


def _build_p8():
    def body(y2_hbm, src_hbm, dst_hbm, eb2_hbm, out_hbm,
             idx_v, rows_y, rows_e, idx_t, rows_yt, rows_et, acc, sem):
        c = lax.axis_index("c")
        s = lax.axis_index("s")
        _init_acc(acc, rows_y, s, 16)
        plsc.subcore_barrier()
        base = (c * NS + s) * EPW

        def chunk(off, n, i_v, r_y, r_e):
            pltpu.sync_copy(src_hbm.at[pl.ds(off, n)], i_v)
            pltpu.async_copy(y2_hbm.at[i_v], r_y, sem).wait()
            pltpu.sync_copy(eb2_hbm.at[pl.ds(off, n)], r_e)

            def mul(i, _):
                r_y[i, :] = r_y[i, :] * r_e[i, :]
                return 0
            lax.fori_loop(0, n, mul, 0)
            pltpu.sync_copy(dst_hbm.at[pl.ds(off, n)], i_v)
            pltpu.sync_copy(r_y, acc.at[i_v], add=True)

        def loop(k, _):
            chunk(base + k * CH, CH, idx_v, rows_y, rows_e)
            return 0
        lax.fori_loop(0, NFULL, loop, 0)
        chunk(base + NFULL * CH, TAIL, idx_t, rows_yt, rows_et)
        plsc.subcore_barrier()
        pltpu.sync_copy(acc.at[pl.ds(s * RPT, RPT)],
                        out_hbm.at[pl.ds(c * N + s * RPT, RPT)])

    return pl.kernel(
        body,
        compiler_params=pltpu.CompilerParams(use_tc_tiling_on_sc=False),
        out_type=_f32((2 * N, 16)),
        mesh=_mesh(),
        scratch_types=[
            pltpu.VMEM((CH,), jnp.int32),
            pltpu.VMEM((CH, 16), jnp.float32), pltpu.VMEM((CH, 16), jnp.float32),
            pltpu.VMEM((TAIL,), jnp.int32),
            pltpu.VMEM((TAIL, 16), jnp.float32), pltpu.VMEM((TAIL, 16), jnp.float32),
            pltpu.VMEM_SHARED((N, 16), jnp.float32),
            pltpu.SemaphoreType.DMA,
        ],
    )


BE = 2000


def _sh_components(u):
    x = u[:, 0:1]
    y = u[:, 1:2]
    z = u[:, 2:3]
    s3 = math.sqrt(3.0)
    s15 = math.sqrt(15.0)
    s5 = math.sqrt(5.0)
    a = math.sqrt(35.0 / 8.0)
    b = math.sqrt(105.0)
    cc = math.sqrt(21.0 / 8.0)
    d = math.sqrt(7.0) / 2.0
    e = math.sqrt(105.0) / 2.0
    x2 = x * x
    y2 = y * y
    z2 = z * z
    comps = [
        jnp.ones_like(x),
        s3 * x, s3 * y, s3 * z,
        s15 * x * y, s15 * y * z, (s5 / 2.0) * (3.0 * z2 - 1.0),
        s15 * x * z, (s15 / 2.0) * (x2 - y2),
        a * y * (3.0 * x2 - y2), b * x * y * z, cc * y * (5.0 * z2 - 1.0),
        d * (5.0 * z2 * z - 3.0 * z), cc * x * (5.0 * z2 - 1.0),
        e * (x2 - y2) * z, a * x * (x2 - 3.0 * y2),
    ]
    return jnp.concatenate(comps, axis=1)


def _soft_onehot_cols(dist):
    cols = []
    for v in (1.0, 1.5, 2.0):
        diff = (dist - v) / 0.5
        m = jnp.abs(diff) < 1.0
        yv = jnp.where(m, jnp.exp(-1.0 / jnp.where(m, 1.0 - diff * diff, 1.0)), 0.0)
        cols.append(KSOH * yv)
    return jnp.concatenate(cols, axis=1)


def _p2_body(ev_ref, w11_ref, w12_ref, w1b_ref, w21_ref, w22_ref, w2b_ref,
             ea_ref, eb1_ref, eb2_ref):
    ev = ev_ref[...]
    n = jnp.sqrt(jnp.sum(ev * ev, axis=1, keepdims=True) + 1e-12)
    u = ev / n
    ea = _sh_components(u)
    el = _soft_onehot_cols(n)
    h1 = jnp.maximum(jnp.dot(el, w11_ref[...], preferred_element_type=jnp.float32), 0.0)
    w = jnp.dot(h1, w12_ref[...], preferred_element_type=jnp.float32)
    eb1 = jnp.dot(ea, w1b_ref[...], preferred_element_type=jnp.float32) * w * INV
    h2 = jnp.maximum(jnp.dot(el, w21_ref[...], preferred_element_type=jnp.float32), 0.0)
    w2 = jnp.dot(h2, w22_ref[...], preferred_element_type=jnp.float32)
    t2 = jnp.dot(ea, w2b_ref[...], preferred_element_type=jnp.float32) * w2
    ea_ref[...] = ea
    eb1_ref[...] = eb1
    eb2_ref[...] = jnp.concatenate(
        [t2, jnp.zeros((t2.shape[0], 13), jnp.float32)], axis=1)


def _build_p2():
    full = lambda shape: pl.BlockSpec(shape, lambda i: (0, 0))
    return pl.pallas_call(
        _p2_body,
        grid=(E // BE,),
        in_specs=[
            pl.BlockSpec((BE, 16), lambda i: (i, 0)),
            full((3, 256)), full((256, 160)), full((16, 160)),
            full((3, 256)), full((256, 3)), full((16, 3)),
        ],
        out_specs=[
            pl.BlockSpec((BE, 16), lambda i: (i, 0)),
            pl.BlockSpec((BE, 160), lambda i: (i, 0)),
            pl.BlockSpec((BE, 16), lambda i: (i, 0)),
        ],
        out_shape=[_f32((E, 16)), _f32((E, 160)), _f32((E, 16))],
    )


def _p5_body(g0_ref, g1_ref, eb1_ref, w1a_ref, msg_ref):
    sfeat = g0_ref[...] + g1_ref[...]
    msg_ref[...] = jnp.dot(sfeat, w1a_ref[...],
                           preferred_element_type=jnp.float32) * eb1_ref[...]


def _build_p5():
    return pl.pallas_call(
        _p5_body,
        grid=(E // BE,),
        in_specs=[
            pl.BlockSpec((BE, 16), lambda i: (i, 0)),
            pl.BlockSpec((BE, 16), lambda i: (i, 0)),
            pl.BlockSpec((BE, 160), lambda i: (i, 0)),
            pl.BlockSpec((16, 160), lambda i: (0, 0)),
        ],
        out_specs=pl.BlockSpec((BE, 160), lambda i: (i, 0)),
        out_shape=_f32((E, 160)),
    )


BN = 2000


def _p7_body(p0_ref, p1_ref, w2a_ref, y2_ref):
    x1 = (p0_ref[...] + p1_ref[...]) * INV
    s_part = jnp.concatenate(
        [jnp.maximum(x1[:, 0:16], 0.0), jnp.abs(x1[:, 16:32])], axis=1)
    g = jnp.concatenate(
        [jnp.maximum(x1[:, 32:40], 0.0), jnp.tanh(x1[:, 40:48]),
         jnp.maximum(x1[:, 48:56], 0.0), jnp.tanh(x1[:, 56:64])], axis=1)
    i0 = lax.broadcasted_iota(jnp.int32, (32, 96), 0)
    i1 = lax.broadcasted_iota(jnp.int32, (32, 96), 1)
    rep = (i1 // 3 == i0).astype(jnp.float32)
    g_exp = jnp.dot(g, rep, preferred_element_type=jnp.float32)
    feat = x1[:, 64:160] * g_exp
    y2 = (jnp.dot(s_part, w2a_ref[0:32, :], preferred_element_type=jnp.float32)
          + jnp.dot(feat, w2a_ref[32:128, :], preferred_element_type=jnp.float32))
    y2_ref[...] = jnp.concatenate(
        [y2, jnp.zeros((y2.shape[0], 13), jnp.float32)], axis=1)


def _build_p7():
    return pl.pallas_call(
        _p7_body,
        grid=(N // BN,),
        in_specs=[
            pl.BlockSpec((BN, 160), lambda i: (i, 0)),
            pl.BlockSpec((BN, 160), lambda i: (i, 0)),
            pl.BlockSpec((128, 3), lambda i: (0, 0)),
        ],
        out_specs=pl.BlockSpec((BN, 16), lambda i: (i, 0)),
        out_shape=_f32((N, 16)),
    )


def _p9_body(p0_ref, p1_ref, out_ref):
    out_ref[...] = ((p0_ref[...] + p1_ref[...]) * INV)[:, 0:3]


def _build_p9():
    return pl.pallas_call(
        _p9_body,
        grid=(N // BN,),
        in_specs=[
            pl.BlockSpec((BN, 16), lambda i: (i, 0)),
            pl.BlockSpec((BN, 16), lambda i: (i, 0)),
        ],
        out_specs=pl.BlockSpec((BN, 3), lambda i: (i, 0)),
        out_shape=_f32((N, 3)),
    )


_P1 = _build_p1()
_P2 = _build_p2()
_P3 = _build_scatter(16)
_P4 = _build_p4()
_P5 = _build_p5()
_P6 = _build_scatter(160)
_P7 = _build_p7()
_P8 = _build_p8()
_P9 = _build_p9()


def kernel(position, edge_src, edge_dst, fc1_w1, fc1_w2, W1a, W1b,
           fc2_w1, fc2_w2, W2a, W2b):
    pos_pad = jnp.concatenate(
        [position, jnp.zeros((N, 13), position.dtype)], axis=1)
    src = edge_src.astype(jnp.int32)
    dst = edge_dst.astype(jnp.int32)

    ev = _P1(pos_pad, src, dst)
    ea, eb1, eb2 = _P2(ev, fc1_w1, fc1_w2, W1b, fc2_w1, fc2_w2, W2b)
    s0 = _P3(ea, dst)
    g0, g1 = _P4(s0[:N], s0[N:], src)
    msg = _P5(g0, g1, eb1, W1a)
    s1 = _P6(msg, dst)
    y2 = _P7(s1[:N], s1[N:], W2a)
    o = _P8(y2, src, dst, eb2)
    return _P9(o[:N], o[N:])


# speedup vs baseline: 2.0709x; 1.1508x over previous
import functools
import math

import jax
import jax.numpy as jnp
from jax import lax
from jax.experimental import pallas as pl
from jax.experimental.pallas import tpu as pltpu
from jax.experimental.pallas import tpu_sc as plsc

N = 10000
E = 160000
NC = 2
NS = 16
NW = NC * NS
EPW = E // NW
CH = 128
NFULL = EPW // CH
TAIL = EPW - NFULL * CH
RPT = N // NS
INV = 1.0 / math.sqrt(3.8)
KSOH = 1.14136 * math.exp(2.0) * math.sqrt(3.0)


def _mesh():
    return plsc.VectorSubcoreMesh(core_axis_name="c", subcore_axis_name="s",
                                  num_cores=NC, num_subcores=NS)


def _f32(shape):
    return jax.ShapeDtypeStruct(shape, jnp.float32)


def _zero_rows(rows_ref, nrows, width):
    def body(i, _):
        for j in range(width // 16):
            rows_ref[i, pl.ds(16 * j, 16)] = jnp.zeros((16,), jnp.float32)
        return 0
    lax.fori_loop(0, nrows, body, 0)


def _init_acc(acc, rows_ref, sid, width):
    _zero_rows(rows_ref, CH, width)
    base = sid * RPT
    off = 0
    while off < RPT:
        n = min(CH, RPT - off)
        pltpu.sync_copy(rows_ref.at[pl.ds(0, n)], acc.at[pl.ds(base + off, n)])
        off += n


def _build_p1():
    def body(pos_hbm, src_hbm, dst_hbm, ev_hbm,
             idx_s, idx_d, rows_s, rows_d, idx_st, idx_dt, rows_st, rows_dt, sem):
        c = lax.axis_index("c")
        s = lax.axis_index("s")
        base = (c * NS + s) * EPW

        def chunk(off, n, i_s, i_d, r_s, r_d):
            pltpu.sync_copy(src_hbm.at[pl.ds(off, n)], i_s)
            pltpu.sync_copy(dst_hbm.at[pl.ds(off, n)], i_d)
            pltpu.async_copy(pos_hbm.at[i_s], r_s, sem).wait()
            pltpu.async_copy(pos_hbm.at[i_d], r_d, sem).wait()

            def sub(i, _):
                r_s[i, :] = r_s[i, :] - r_d[i, :]
                return 0
            lax.fori_loop(0, n, sub, 0)
            pltpu.sync_copy(r_s, ev_hbm.at[pl.ds(off, n)])

        def loop(k, _):
            chunk(base + k * CH, CH, idx_s, idx_d, rows_s, rows_d)
            return 0
        lax.fori_loop(0, NFULL, loop, 0)
        chunk(base + NFULL * CH, TAIL, idx_st, idx_dt, rows_st, rows_dt)

    return pl.kernel(
        body,
        compiler_params=pltpu.CompilerParams(use_tc_tiling_on_sc=False),
        out_type=_f32((E, 16)),
        mesh=_mesh(),
        scratch_types=[
            pltpu.VMEM((CH,), jnp.int32), pltpu.VMEM((CH,), jnp.int32),
            pltpu.VMEM((CH, 16), jnp.float32), pltpu.VMEM((CH, 16), jnp.float32),
            pltpu.VMEM((TAIL,), jnp.int32), pltpu.VMEM((TAIL,), jnp.int32),
            pltpu.VMEM((TAIL, 16), jnp.float32), pltpu.VMEM((TAIL, 16), jnp.float32),
            pltpu.SemaphoreType.DMA,
        ],
    )


def _build_scatter(D):
    def body(val_hbm, dst_hbm, out_hbm, idx_v, rows_v, idx_t, rows_t, acc):
        c = lax.axis_index("c")
        s = lax.axis_index("s")
        _init_acc(acc, rows_v, s, D)
        plsc.subcore_barrier()
        base = (c * NS + s) * EPW

        def chunk(off, n, i_v, r_v):
            pltpu.sync_copy(dst_hbm.at[pl.ds(off, n)], i_v)
            pltpu.sync_copy(val_hbm.at[pl.ds(off, n)], r_v)
            pltpu.sync_copy(r_v, acc.at[i_v], add=True)

        def loop(k, _):
            chunk(base + k * CH, CH, idx_v, rows_v)
            return 0
        lax.fori_loop(0, NFULL, loop, 0)
        chunk(base + NFULL * CH, TAIL, idx_t, rows_t)
        plsc.subcore_barrier()
        pltpu.sync_copy(acc.at[pl.ds(s * RPT, RPT)],
                        out_hbm.at[pl.ds(c * N + s * RPT, RPT)])

    return pl.kernel(
        body,
        compiler_params=pltpu.CompilerParams(use_tc_tiling_on_sc=False),
        out_type=_f32((2 * N, D)),
        mesh=_mesh(),
        scratch_types=[
            pltpu.VMEM((CH,), jnp.int32),
            pltpu.VMEM((CH, D), jnp.float32),
            pltpu.VMEM((TAIL,), jnp.int32),
            pltpu.VMEM((TAIL, D), jnp.float32),
            pltpu.VMEM_SHARED((N, D), jnp.float32),
        ],
    )


def _build_p4():
    def body(p0_hbm, p1_hbm, src_hbm, g0_hbm, g1_hbm,
             idx_v, rows_v, idx_t, rows_t, sem):
        c = lax.axis_index("c")
        s = lax.axis_index("s")
        base = (c * NS + s) * EPW

        def chunk(off, n, i_v, r_v):
            pltpu.sync_copy(src_hbm.at[pl.ds(off, n)], i_v)
            pltpu.async_copy(p0_hbm.at[i_v], r_v, sem).wait()
            pltpu.sync_copy(r_v, g0_hbm.at[pl.ds(off, n)])
            pltpu.async_copy(p1_hbm.at[i_v], r_v, sem).wait()
            pltpu.sync_copy(r_v, g1_hbm.at[pl.ds(off, n)])

        def loop(k, _):
            chunk(base + k * CH, CH, idx_v, rows_v)
            return 0
        lax.fori_loop(0, NFULL, loop, 0)
        chunk(base + NFULL * CH, TAIL, idx_t, rows_t)

    return pl.kernel(
        body,
        compiler_params=pltpu.CompilerParams(use_tc_tiling_on_sc=False),
        out_type=(_f32((E, 16)), _f32((E, 16))),
        mesh=_mesh(),
        scratch_types=[
            pltpu.VMEM((CH,), jnp.int32), pltpu.VMEM((CH, 16), jnp.float32),
            pltpu.VMEM((TAIL,), jnp.int32), pltpu.VMEM((TAIL, 16), jnp.float32),
            pltpu.SemaphoreType.DMA,
        ],
    )


def _build_p8():
    def body(y2_hbm, src_hbm, dst_hbm, eb2_hbm, out_hbm,
             idx_v, rows_y, rows_e, idx_t, rows_yt, rows_et, acc, sem):
        c = lax.axis_index("c")
        s = lax.axis_index("s")
        _init_acc(acc, rows_y, s, 16)
        plsc.subcore_barrier()
        base = (c * NS + s) * EPW

        def chunk(off, n, i_v, r_y, r_e):
            pltpu.sync_copy(src_hbm.at[pl.ds(off, n)], i_v)
            pltpu.async_copy(y2_hbm.at[i_v], r_y, sem).wait()
            pltpu.sync_copy(eb2_hbm.at[pl.ds(off, n)], r_e)

            def mul(i, _):
                r_y[i, :] = r_y[i, :] * r_e[i, :]
                return 0
            lax.fori_loop(0, n, mul, 0)
            pltpu.sync_copy(dst_hbm.at[pl.ds(off, n)], i_v)
            pltpu.sync_copy(r_y, acc.at[i_v], add=True)

        def loop(k, _):
            chunk(base + k * CH, CH, idx_v, rows_y, rows_e)
            return 0
        lax.fori_loop(0, NFULL, loop, 0)
        chunk(base + NFULL * CH, TAIL, idx_t, rows_yt, rows_et)
        plsc.subcore_barrier()
        pltpu.sync_copy(acc.at[pl.ds(s * RPT, RPT)],
                        out_hbm.at[pl.ds(c * N + s * RPT, RPT)])

    return pl.kernel(
        body,
        compiler_params=pltpu.CompilerParams(use_tc_tiling_on_sc=False),
        out_type=_f32((2 * N, 16)),
        mesh=_mesh(),
        scratch_types=[
            pltpu.VMEM((CH,), jnp.int32),
            pltpu.VMEM((CH, 16), jnp.float32), pltpu.VMEM((CH, 16), jnp.float32),
            pltpu.VMEM((TAIL,), jnp.int32),
            pltpu.VMEM((TAIL, 16), jnp.float32), pltpu.VMEM((TAIL, 16), jnp.float32),
            pltpu.VMEM_SHARED((N, 16), jnp.float32),
            pltpu.SemaphoreType.DMA,
        ],
    )


BE = 2000


def _sh_factor_tables():
    import numpy as np
    s3 = math.sqrt(3.0)
    s15 = math.sqrt(15.0)
    s5 = math.sqrt(5.0)
    a = math.sqrt(35.0 / 8.0)
    b = math.sqrt(105.0)
    c = math.sqrt(21.0 / 8.0)
    d = math.sqrt(7.0) / 2.0
    e = math.sqrt(105.0) / 2.0
    F = [
        ([], 1., [], 1., [], 1.),
        ([(0, s3)], 0., [], 1., [], 1.),
        ([(1, s3)], 0., [], 1., [], 1.),
        ([(2, s3)], 0., [], 1., [], 1.),
        ([(0, s15)], 0., [(1, 1)], 0., [], 1.),
        ([(1, s15)], 0., [(2, 1)], 0., [], 1.),
        ([(2, s5 / 2 * s3)], -s5 / 2, [(2, s3)], 1., [], 1.),
        ([(0, s15)], 0., [(2, 1)], 0., [], 1.),
        ([(0, s15 / 2), (1, -s15 / 2)], 0., [(0, 1), (1, 1)], 0., [], 1.),
        ([(1, a)], 0., [(0, s3), (1, -1)], 0., [(0, s3), (1, 1)], 0.),
        ([(0, b)], 0., [(1, 1)], 0., [(2, 1)], 0.),
        ([(1, c)], 0., [(2, s5)], -1., [(2, s5)], 1.),
        ([(2, d)], 0., [(2, s5)], -s3, [(2, s5)], s3),
        ([(0, c)], 0., [(2, s5)], -1., [(2, s5)], 1.),
        ([(0, e), (1, -e)], 0., [(0, 1), (1, 1)], 0., [(2, 1)], 0.),
        ([(0, a)], 0., [(0, 1), (1, -s3)], 0., [(0, 1), (1, s3)], 0.),
    ]
    A = np.zeros((3, 16, 16), np.float32)
    B = np.zeros((3, 16), np.float32)
    for col, parts in enumerate(F):
        for k in range(3):
            B[k, col] = parts[2 * k + 1]
            for r, coef in parts[2 * k]:
                A[k, r, col] = coef
    return A, B


_SH_A, _SH_B = _sh_factor_tables()


def _soh_rows():
    import numpy as np
    rows = np.zeros((2, 16), np.float32)
    rows[0, :] = 1e9
    rows[0, :3] = [2.0, 3.0, 4.0]
    rows[1, :3] = KSOH
    return rows


_SOH = _soh_rows()


def _p2_body(ev_ref, w11_ref, w12_ref, w1b_ref, w21_ref, w22_ref, w2b_ref,
             a0_ref, a1_ref, a2_ref, b_ref, soh_ref,
             ea_ref, eb1_ref, eb2_ref):
    ev = ev_ref[...]
    n2 = jnp.sum(ev * ev, axis=1, keepdims=True) + 1e-12
    nb = jnp.sqrt(jnp.broadcast_to(n2, ev.shape))
    u = ev / nb
    ea = ((jnp.dot(u, a0_ref[...], preferred_element_type=jnp.float32, precision=jax.lax.Precision.HIGHEST) + b_ref[0:1, :])
          * (jnp.dot(u, a1_ref[...], preferred_element_type=jnp.float32, precision=jax.lax.Precision.HIGHEST) + b_ref[1:2, :])
          * (jnp.dot(u, a2_ref[...], preferred_element_type=jnp.float32, precision=jax.lax.Precision.HIGHEST) + b_ref[2:3, :]))
    diff = 2.0 * nb - soh_ref[0:1, :]
    m = jnp.abs(diff) < 1.0
    yv = jnp.where(m, jnp.exp(-1.0 / jnp.where(m, 1.0 - diff * diff, 1.0)), 0.0)
    el = yv * soh_ref[1:2, :]
    w11p = jnp.concatenate([w11_ref[...], jnp.zeros((13, 256), jnp.float32)], axis=0)
    h1 = jnp.maximum(jnp.dot(el, w11p, preferred_element_type=jnp.float32), 0.0)
    w = jnp.dot(h1, w12_ref[...], preferred_element_type=jnp.float32)
    eb1 = jnp.dot(ea, w1b_ref[...], preferred_element_type=jnp.float32) * w * INV
    w21p = jnp.concatenate([w21_ref[...], jnp.zeros((13, 256), jnp.float32)], axis=0)
    h2 = jnp.maximum(jnp.dot(el, w21p, preferred_element_type=jnp.float32), 0.0)
    w2 = jnp.dot(h2, w22_ref[...], preferred_element_type=jnp.float32)
    t2 = jnp.dot(ea, w2b_ref[...], preferred_element_type=jnp.float32) * w2
    ea_ref[...] = ea
    eb1_ref[...] = eb1
    eb2_ref[...] = jnp.concatenate(
        [t2, jnp.zeros((t2.shape[0], 13), jnp.float32)], axis=1)


def _build_p2():
    full = lambda shape: pl.BlockSpec(shape, lambda i: (0, 0))
    return pl.pallas_call(
        _p2_body,
        grid=(E // BE,),
        in_specs=[
            pl.BlockSpec((BE, 16), lambda i: (i, 0)),
            full((3, 256)), full((256, 160)), full((16, 160)),
            full((3, 256)), full((256, 3)), full((16, 3)),
            full((16, 16)), full((16, 16)), full((16, 16)),
            full((3, 16)), full((2, 16)),
        ],
        out_specs=[
            pl.BlockSpec((BE, 16), lambda i: (i, 0)),
            pl.BlockSpec((BE, 160), lambda i: (i, 0)),
            pl.BlockSpec((BE, 16), lambda i: (i, 0)),
        ],
        out_shape=[_f32((E, 16)), _f32((E, 160)), _f32((E, 16))],
    )


def _p5_body(g0_ref, g1_ref, eb1_ref, w1a_ref, msg_ref):
    sfeat = g0_ref[...] + g1_ref[...]
    msg_ref[...] = jnp.dot(sfeat, w1a_ref[...],
                           preferred_element_type=jnp.float32) * eb1_ref[...]


def _build_p5():
    return pl.pallas_call(
        _p5_body,
        grid=(E // BE,),
        in_specs=[
            pl.BlockSpec((BE, 16), lambda i: (i, 0)),
            pl.BlockSpec((BE, 16), lambda i: (i, 0)),
            pl.BlockSpec((BE, 160), lambda i: (i, 0)),
            pl.BlockSpec((16, 160), lambda i: (0, 0)),
        ],
        out_specs=pl.BlockSpec((BE, 160), lambda i: (i, 0)),
        out_shape=_f32((E, 160)),
    )


BN = 2000


def _p7_body(p0_ref, p1_ref, w2a_ref, y2_ref):
    x1 = (p0_ref[...] + p1_ref[...]) * INV
    s_part = jnp.concatenate(
        [jnp.maximum(x1[:, 0:16], 0.0), jnp.abs(x1[:, 16:32])], axis=1)
    g = jnp.concatenate(
        [jnp.maximum(x1[:, 32:40], 0.0), jnp.tanh(x1[:, 40:48]),
         jnp.maximum(x1[:, 48:56], 0.0), jnp.tanh(x1[:, 56:64])], axis=1)
    i0 = lax.broadcasted_iota(jnp.int32, (32, 96), 0)
    i1 = lax.broadcasted_iota(jnp.int32, (32, 96), 1)
    rep = (i1 // 3 == i0).astype(jnp.float32)
    g_exp = jnp.dot(g, rep, preferred_element_type=jnp.float32)
    feat = x1[:, 64:160] * g_exp
    y2 = (jnp.dot(s_part, w2a_ref[0:32, :], preferred_element_type=jnp.float32)
          + jnp.dot(feat, w2a_ref[32:128, :], preferred_element_type=jnp.float32))
    y2_ref[...] = jnp.concatenate(
        [y2, jnp.zeros((y2.shape[0], 13), jnp.float32)], axis=1)


def _build_p7():
    return pl.pallas_call(
        _p7_body,
        grid=(N // BN,),
        in_specs=[
            pl.BlockSpec((BN, 160), lambda i: (i, 0)),
            pl.BlockSpec((BN, 160), lambda i: (i, 0)),
            pl.BlockSpec((128, 3), lambda i: (0, 0)),
        ],
        out_specs=pl.BlockSpec((BN, 16), lambda i: (i, 0)),
        out_shape=_f32((N, 16)),
    )


def _p9_body(p0_ref, p1_ref, out_ref):
    out_ref[...] = ((p0_ref[...] + p1_ref[...]) * INV)[:, 0:3]


def _build_p9():
    return pl.pallas_call(
        _p9_body,
        grid=(N // BN,),
        in_specs=[
            pl.BlockSpec((BN, 16), lambda i: (i, 0)),
            pl.BlockSpec((BN, 16), lambda i: (i, 0)),
        ],
        out_specs=pl.BlockSpec((BN, 3), lambda i: (i, 0)),
        out_shape=_f32((N, 3)),
    )


_P1 = _build_p1()
_P2 = _build_p2()
_P3 = _build_scatter(16)
_P4 = _build_p4()
_P5 = _build_p5()
_P6 = _build_scatter(160)
_P7 = _build_p7()
_P8 = _build_p8()
_P9 = _build_p9()


def kernel(position, edge_src, edge_dst, fc1_w1, fc1_w2, W1a, W1b,
           fc2_w1, fc2_w2, W2a, W2b):
    pos_pad = jnp.concatenate(
        [position, jnp.zeros((N, 13), position.dtype)], axis=1)
    src = edge_src.astype(jnp.int32)
    dst = edge_dst.astype(jnp.int32)

    ev = _P1(pos_pad, src, dst)
    ea, eb1, eb2 = _P2(ev, fc1_w1, fc1_w2, W1b, fc2_w1, fc2_w2, W2b,
                       _SH_A[0], _SH_A[1], _SH_A[2], _SH_B, _SOH)
    s0 = _P3(ea, dst)
    g0, g1 = _P4(s0[:N], s0[N:], src)
    msg = _P5(g0, g1, eb1, W1a)
    s1 = _P6(msg, dst)
    y2 = _P7(s1[:N], s1[N:], W2a)
    o = _P8(y2, src, dst, eb2)
    return _P9(o[:N], o[N:])


# speedup vs baseline: 2.1436x; 1.0351x over previous
import functools
import math

import jax
import jax.numpy as jnp
from jax import lax
from jax.experimental import pallas as pl
from jax.experimental.pallas import tpu as pltpu
from jax.experimental.pallas import tpu_sc as plsc

N = 10000
E = 160000
NC = 2
NS = 16
NW = NC * NS
EPW = E // NW
CH = 128
NFULL = EPW // CH
TAIL = EPW - NFULL * CH
RPT = N // NS
INV = 1.0 / math.sqrt(3.8)
KSOH = 1.14136 * math.exp(2.0) * math.sqrt(3.0)


def _mesh():
    return plsc.VectorSubcoreMesh(core_axis_name="c", subcore_axis_name="s",
                                  num_cores=NC, num_subcores=NS)


def _f32(shape):
    return jax.ShapeDtypeStruct(shape, jnp.float32)


def _zero_rows(rows_ref, nrows, width):
    def body(i, _):
        for j in range(width // 16):
            rows_ref[i, pl.ds(16 * j, 16)] = jnp.zeros((16,), jnp.float32)
        return 0
    lax.fori_loop(0, nrows, body, 0)


def _init_acc(acc, rows_ref, sid, width):
    _zero_rows(rows_ref, CH, width)
    base = sid * RPT
    off = 0
    while off < RPT:
        n = min(CH, RPT - off)
        pltpu.sync_copy(rows_ref.at[pl.ds(0, n)], acc.at[pl.ds(base + off, n)])
        off += n


def _build_p1():
    def body(pos_hbm, src_hbm, dst_hbm, ev_hbm,
             idx_s, idx_d, rows_s, rows_d, idx_st, idx_dt, rows_st, rows_dt,
             sem, sem2):
        c = lax.axis_index("c")
        s = lax.axis_index("s")
        base = (c * NS + s) * EPW

        def chunk(off, n, i_s, i_d, r_s, r_d):
            di = pltpu.async_copy(src_hbm.at[pl.ds(off, n)], i_s, sem)
            dd = pltpu.async_copy(dst_hbm.at[pl.ds(off, n)], i_d, sem2)
            di.wait()
            dd.wait()
            gs = pltpu.async_copy(pos_hbm.at[i_s], r_s, sem)
            gd = pltpu.async_copy(pos_hbm.at[i_d], r_d, sem2)
            gs.wait()
            gd.wait()

            def sub(i, _):
                r_s[i, :] = r_s[i, :] - r_d[i, :]
                return 0
            lax.fori_loop(0, n, sub, 0)
            pltpu.sync_copy(r_s, ev_hbm.at[pl.ds(off, n)])

        def loop(k, _):
            chunk(base + k * CH, CH, idx_s, idx_d, rows_s, rows_d)
            return 0
        lax.fori_loop(0, NFULL, loop, 0)
        chunk(base + NFULL * CH, TAIL, idx_st, idx_dt, rows_st, rows_dt)

    return pl.kernel(
        body,
        compiler_params=pltpu.CompilerParams(use_tc_tiling_on_sc=False),
        out_type=_f32((E, 16)),
        mesh=_mesh(),
        scratch_types=[
            pltpu.VMEM((CH,), jnp.int32), pltpu.VMEM((CH,), jnp.int32),
            pltpu.VMEM((CH, 16), jnp.float32), pltpu.VMEM((CH, 16), jnp.float32),
            pltpu.VMEM((TAIL,), jnp.int32), pltpu.VMEM((TAIL,), jnp.int32),
            pltpu.VMEM((TAIL, 16), jnp.float32), pltpu.VMEM((TAIL, 16), jnp.float32),
            pltpu.SemaphoreType.DMA, pltpu.SemaphoreType.DMA,
        ],
    )


def _build_scatter(D):
    def body(val_hbm, dst_hbm, out_hbm, idx_v, rows_v, idx_t, rows_t, acc,
             sem_i, sem_r):
        c = lax.axis_index("c")
        s = lax.axis_index("s")
        _init_acc(acc, rows_v, s, D)
        plsc.subcore_barrier()
        base = (c * NS + s) * EPW

        def chunk(off, n, i_v, r_v):
            di = pltpu.async_copy(dst_hbm.at[pl.ds(off, n)], i_v, sem_i)
            dr = pltpu.async_copy(val_hbm.at[pl.ds(off, n)], r_v, sem_r)
            di.wait()
            dr.wait()
            pltpu.sync_copy(r_v, acc.at[i_v], add=True)

        def loop(k, _):
            chunk(base + k * CH, CH, idx_v, rows_v)
            return 0
        lax.fori_loop(0, NFULL, loop, 0)
        chunk(base + NFULL * CH, TAIL, idx_t, rows_t)
        plsc.subcore_barrier()
        pltpu.sync_copy(acc.at[pl.ds(s * RPT, RPT)],
                        out_hbm.at[pl.ds(c * N + s * RPT, RPT)])

    return pl.kernel(
        body,
        compiler_params=pltpu.CompilerParams(use_tc_tiling_on_sc=False),
        out_type=_f32((2 * N, D)),
        mesh=_mesh(),
        scratch_types=[
            pltpu.VMEM((CH,), jnp.int32),
            pltpu.VMEM((CH, D), jnp.float32),
            pltpu.VMEM((TAIL,), jnp.int32),
            pltpu.VMEM((TAIL, D), jnp.float32),
            pltpu.VMEM_SHARED((N, D), jnp.float32),
            pltpu.SemaphoreType.DMA, pltpu.SemaphoreType.DMA,
        ],
    )


GCH = 128
GPT = E // NS
GFULL = GPT // GCH
GTAIL = GPT - GFULL * GCH


def _build_p34():
    def body(ea_hbm, src_hbm, dst_hbm, g_hbm,
             idx_v, rows_v, idx_t8, rows_t8, idx_t16, rows_t16, acc,
             sem_i, sem_r):
        c = lax.axis_index("c")
        s = lax.axis_index("s")
        _init_acc(acc, rows_v, s, 16)
        plsc.subcore_barrier()
        base = (c * NS + s) * EPW

        def sc_chunk(off, n, i_v, r_v):
            di = pltpu.async_copy(dst_hbm.at[pl.ds(off, n)], i_v, sem_i)
            dr = pltpu.async_copy(ea_hbm.at[pl.ds(off, n)], r_v, sem_r)
            di.wait()
            dr.wait()
            pltpu.sync_copy(r_v, acc.at[i_v], add=True)

        def sloop(k, _):
            sc_chunk(base + k * CH, CH, idx_v, rows_v)
            return 0
        lax.fori_loop(0, NFULL, sloop, 0)
        sc_chunk(base + NFULL * CH, TAIL, idx_t8, rows_t8)
        plsc.subcore_barrier()

        gbase = s * GPT

        def g_chunk(off, n, i_v, r_v):
            pltpu.async_copy(src_hbm.at[pl.ds(off, n)], i_v, sem_i).wait()
            pltpu.async_copy(acc.at[i_v], r_v, sem_r).wait()
            pltpu.sync_copy(r_v, g_hbm.at[pl.ds(c * E + off, n)])

        def gloop(k, _):
            g_chunk(gbase + k * GCH, GCH, idx_v, rows_v)
            return 0
        lax.fori_loop(0, GFULL, gloop, 0)
        g_chunk(gbase + GFULL * GCH, GTAIL, idx_t16, rows_t16)

    return pl.kernel(
        body,
        compiler_params=pltpu.CompilerParams(use_tc_tiling_on_sc=False),
        out_type=_f32((2 * E, 16)),
        mesh=_mesh(),
        scratch_types=[
            pltpu.VMEM((CH,), jnp.int32), pltpu.VMEM((CH, 16), jnp.float32),
            pltpu.VMEM((TAIL,), jnp.int32), pltpu.VMEM((TAIL, 16), jnp.float32),
            pltpu.VMEM((GTAIL,), jnp.int32), pltpu.VMEM((GTAIL, 16), jnp.float32),
            pltpu.VMEM_SHARED((N, 16), jnp.float32),
            pltpu.SemaphoreType.DMA, pltpu.SemaphoreType.DMA,
        ],
    )


def _build_p8():
    def body(y2_hbm, src_hbm, dst_hbm, eb2_hbm, out_hbm,
             idx_v, rows_y, rows_e, idx_t, rows_yt, rows_et, acc, sem):
        c = lax.axis_index("c")
        s = lax.axis_index("s")
        _init_acc(acc, rows_y, s, 16)
        plsc.subcore_barrier()
        base = (c * NS + s) * EPW

        def chunk(off, n, i_v, r_y, r_e):
            pltpu.sync_copy(src_hbm.at[pl.ds(off, n)], i_v)
            pltpu.async_copy(y2_hbm.at[i_v], r_y, sem).wait()
            pltpu.sync_copy(eb2_hbm.at[pl.ds(off, n)], r_e)

            def mul(i, _):
                r_y[i, :] = r_y[i, :] * r_e[i, :]
                return 0
            lax.fori_loop(0, n, mul, 0)
            pltpu.sync_copy(dst_hbm.at[pl.ds(off, n)], i_v)
            pltpu.sync_copy(r_y, acc.at[i_v], add=True)

        def loop(k, _):
            chunk(base + k * CH, CH, idx_v, rows_y, rows_e)
            return 0
        lax.fori_loop(0, NFULL, loop, 0)
        chunk(base + NFULL * CH, TAIL, idx_t, rows_yt, rows_et)
        plsc.subcore_barrier()
        pltpu.sync_copy(acc.at[pl.ds(s * RPT, RPT)],
                        out_hbm.at[pl.ds(c * N + s * RPT, RPT)])

    return pl.kernel(
        body,
        compiler_params=pltpu.CompilerParams(use_tc_tiling_on_sc=False),
        out_type=_f32((2 * N, 16)),
        mesh=_mesh(),
        scratch_types=[
            pltpu.VMEM((CH,), jnp.int32),
            pltpu.VMEM((CH, 16), jnp.float32), pltpu.VMEM((CH, 16), jnp.float32),
            pltpu.VMEM((TAIL,), jnp.int32),
            pltpu.VMEM((TAIL, 16), jnp.float32), pltpu.VMEM((TAIL, 16), jnp.float32),
            pltpu.VMEM_SHARED((N, 16), jnp.float32),
            pltpu.SemaphoreType.DMA,
        ],
    )


BE = 2000


def _sh_factor_tables():
    import numpy as np
    s3 = math.sqrt(3.0)
    s15 = math.sqrt(15.0)
    s5 = math.sqrt(5.0)
    a = math.sqrt(35.0 / 8.0)
    b = math.sqrt(105.0)
    c = math.sqrt(21.0 / 8.0)
    d = math.sqrt(7.0) / 2.0
    e = math.sqrt(105.0) / 2.0
    F = [
        ([], 1., [], 1., [], 1.),
        ([(0, s3)], 0., [], 1., [], 1.),
        ([(1, s3)], 0., [], 1., [], 1.),
        ([(2, s3)], 0., [], 1., [], 1.),
        ([(0, s15)], 0., [(1, 1)], 0., [], 1.),
        ([(1, s15)], 0., [(2, 1)], 0., [], 1.),
        ([(2, s5 / 2 * s3)], -s5 / 2, [(2, s3)], 1., [], 1.),
        ([(0, s15)], 0., [(2, 1)], 0., [], 1.),
        ([(0, s15 / 2), (1, -s15 / 2)], 0., [(0, 1), (1, 1)], 0., [], 1.),
        ([(1, a)], 0., [(0, s3), (1, -1)], 0., [(0, s3), (1, 1)], 0.),
        ([(0, b)], 0., [(1, 1)], 0., [(2, 1)], 0.),
        ([(1, c)], 0., [(2, s5)], -1., [(2, s5)], 1.),
        ([(2, d)], 0., [(2, s5)], -s3, [(2, s5)], s3),
        ([(0, c)], 0., [(2, s5)], -1., [(2, s5)], 1.),
        ([(0, e), (1, -e)], 0., [(0, 1), (1, 1)], 0., [(2, 1)], 0.),
        ([(0, a)], 0., [(0, 1), (1, -s3)], 0., [(0, 1), (1, s3)], 0.),
    ]
    A = np.zeros((3, 16, 16), np.float32)
    B = np.zeros((3, 16), np.float32)
    for col, parts in enumerate(F):
        for k in range(3):
            B[k, col] = parts[2 * k + 1]
            for r, coef in parts[2 * k]:
                A[k, r, col] = coef
    return A, B


_SH_A, _SH_B = _sh_factor_tables()


def _soh_rows():
    import numpy as np
    rows = np.zeros((2, 16), np.float32)
    rows[0, :] = 1e9
    rows[0, :3] = [2.0, 3.0, 4.0]
    rows[1, :3] = KSOH
    return rows


_SOH = _soh_rows()


def _p2_body(ev_ref, w11_ref, w12_ref, w1b_ref, w21_ref, w22_ref, w2b_ref,
             a0_ref, a1_ref, a2_ref, b_ref, soh_ref,
             ea_ref, eb1_ref, eb2_ref):
    ev = ev_ref[...]
    n2 = jnp.sum(ev * ev, axis=1, keepdims=True) + 1e-12
    nb = jnp.sqrt(jnp.broadcast_to(n2, ev.shape))
    u = ev / nb
    ea = ((jnp.dot(u, a0_ref[...], preferred_element_type=jnp.float32, precision=jax.lax.Precision.HIGHEST) + b_ref[0:1, :])
          * (jnp.dot(u, a1_ref[...], preferred_element_type=jnp.float32, precision=jax.lax.Precision.HIGHEST) + b_ref[1:2, :])
          * (jnp.dot(u, a2_ref[...], preferred_element_type=jnp.float32, precision=jax.lax.Precision.HIGHEST) + b_ref[2:3, :]))
    diff = 2.0 * nb - soh_ref[0:1, :]
    m = jnp.abs(diff) < 1.0
    yv = jnp.where(m, jnp.exp(-1.0 / jnp.where(m, 1.0 - diff * diff, 1.0)), 0.0)
    el = yv * soh_ref[1:2, :]
    w11p = jnp.concatenate([w11_ref[...], jnp.zeros((13, 256), jnp.float32)], axis=0)
    h1 = jnp.maximum(jnp.dot(el, w11p, preferred_element_type=jnp.float32), 0.0)
    w = jnp.dot(h1, w12_ref[...], preferred_element_type=jnp.float32)
    eb1 = jnp.dot(ea, w1b_ref[...], preferred_element_type=jnp.float32) * w * INV
    w21p = jnp.concatenate([w21_ref[...], jnp.zeros((13, 256), jnp.float32)], axis=0)
    h2 = jnp.maximum(jnp.dot(el, w21p, preferred_element_type=jnp.float32), 0.0)
    w2 = jnp.dot(h2, w22_ref[...], preferred_element_type=jnp.float32)
    t2 = jnp.dot(ea, w2b_ref[...], preferred_element_type=jnp.float32) * w2
    ea_ref[...] = ea
    eb1_ref[...] = eb1.astype(jnp.bfloat16)
    eb2_ref[...] = jnp.concatenate(
        [t2, jnp.zeros((t2.shape[0], 13), jnp.float32)], axis=1)


def _build_p2():
    full = lambda shape: pl.BlockSpec(shape, lambda i: (0, 0))
    return pl.pallas_call(
        _p2_body,
        grid=(E // BE,),
        in_specs=[
            pl.BlockSpec((BE, 16), lambda i: (i, 0)),
            full((3, 256)), full((256, 160)), full((16, 160)),
            full((3, 256)), full((256, 3)), full((16, 3)),
            full((16, 16)), full((16, 16)), full((16, 16)),
            full((3, 16)), full((2, 16)),
        ],
        out_specs=[
            pl.BlockSpec((BE, 16), lambda i: (i, 0)),
            pl.BlockSpec((BE, 160), lambda i: (i, 0)),
            pl.BlockSpec((BE, 16), lambda i: (i, 0)),
        ],
        out_shape=[_f32((E, 16)),
                   jax.ShapeDtypeStruct((E, 160), jnp.bfloat16),
                   _f32((E, 16))],
    )


def _p5_body(g0_ref, g1_ref, eb1_ref, w1a_ref, msg_ref):
    sfeat = g0_ref[...] + g1_ref[...]
    msg_ref[...] = jnp.dot(sfeat, w1a_ref[...],
                           preferred_element_type=jnp.float32) \
        * eb1_ref[...].astype(jnp.float32)


def _build_p5():
    return pl.pallas_call(
        _p5_body,
        grid=(E // BE,),
        in_specs=[
            pl.BlockSpec((BE, 16), lambda i: (i, 0)),
            pl.BlockSpec((BE, 16), lambda i: (i, 0)),
            pl.BlockSpec((BE, 160), lambda i: (i, 0)),
            pl.BlockSpec((16, 160), lambda i: (0, 0)),
        ],
        out_specs=pl.BlockSpec((BE, 160), lambda i: (i, 0)),
        out_shape=_f32((E, 160)),
    )


BN = 2000


def _p7_body(p0_ref, p1_ref, w2a_ref, y2_ref):
    x1 = (p0_ref[...] + p1_ref[...]) * INV
    s_part = jnp.concatenate(
        [jnp.maximum(x1[:, 0:16], 0.0), jnp.abs(x1[:, 16:32])], axis=1)
    g = jnp.concatenate(
        [jnp.maximum(x1[:, 32:40], 0.0), jnp.tanh(x1[:, 40:48]),
         jnp.maximum(x1[:, 48:56], 0.0), jnp.tanh(x1[:, 56:64])], axis=1)
    i0 = lax.broadcasted_iota(jnp.int32, (32, 96), 0)
    i1 = lax.broadcasted_iota(jnp.int32, (32, 96), 1)
    rep = (i1 // 3 == i0).astype(jnp.float32)
    g_exp = jnp.dot(g, rep, preferred_element_type=jnp.float32)
    feat = x1[:, 64:160] * g_exp
    y2 = (jnp.dot(s_part, w2a_ref[0:32, :], preferred_element_type=jnp.float32)
          + jnp.dot(feat, w2a_ref[32:128, :], preferred_element_type=jnp.float32))
    y2_ref[...] = jnp.concatenate(
        [y2, jnp.zeros((y2.shape[0], 13), jnp.float32)], axis=1)


def _build_p7():
    return pl.pallas_call(
        _p7_body,
        grid=(N // BN,),
        in_specs=[
            pl.BlockSpec((BN, 160), lambda i: (i, 0)),
            pl.BlockSpec((BN, 160), lambda i: (i, 0)),
            pl.BlockSpec((128, 3), lambda i: (0, 0)),
        ],
        out_specs=pl.BlockSpec((BN, 16), lambda i: (i, 0)),
        out_shape=_f32((N, 16)),
    )


def _p9_body(p0_ref, p1_ref, out_ref):
    out_ref[...] = ((p0_ref[...] + p1_ref[...]) * INV)[:, 0:3]


def _build_p9():
    return pl.pallas_call(
        _p9_body,
        grid=(N // BN,),
        in_specs=[
            pl.BlockSpec((BN, 16), lambda i: (i, 0)),
            pl.BlockSpec((BN, 16), lambda i: (i, 0)),
        ],
        out_specs=pl.BlockSpec((BN, 3), lambda i: (i, 0)),
        out_shape=_f32((N, 3)),
    )


_P1 = _build_p1()
_P2 = _build_p2()
_P34 = _build_p34()
_P5 = _build_p5()
_P6 = _build_scatter(160)
_P7 = _build_p7()
_P8 = _build_p8()
_P9 = _build_p9()


def kernel(position, edge_src, edge_dst, fc1_w1, fc1_w2, W1a, W1b,
           fc2_w1, fc2_w2, W2a, W2b):
    pos_pad = jnp.concatenate(
        [position, jnp.zeros((N, 13), position.dtype)], axis=1)
    src = edge_src.astype(jnp.int32)
    dst = edge_dst.astype(jnp.int32)

    ev = _P1(pos_pad, src, dst)
    ea, eb1, eb2 = _P2(ev, fc1_w1, fc1_w2, W1b, fc2_w1, fc2_w2, W2b,
                       _SH_A[0], _SH_A[1], _SH_A[2], _SH_B, _SOH)
    g = _P34(ea, src, dst)
    msg = _P5(g[:E], g[E:], eb1, W1a)
    s1 = _P6(msg, dst)
    y2 = _P7(s1[:N], s1[N:], W2a)
    o = _P8(y2, src, dst, eb2)
    return _P9(o[:N], o[N:])
